# Initial kernel scaffold; baseline (speedup 1.0000x reference)
import jax
import jax.numpy as jnp
from jax.experimental import pallas as pl


def kernel(tokens, table):
    raise NotImplementedError("write your pallas kernel here")


# speedup vs baseline: 4.6642x; 4.6642x over previous
import functools
import math

import jax
import jax.numpy as jnp
from jax import lax
from jax.experimental import pallas as pl
from jax.experimental.pallas import tpu as pltpu
from jax.experimental.pallas import tpu_sc as plsc

_SEQ, _BATCH, _EMB = 200, 1024, 128
_N = _SEQ * _BATCH
_NC, _NS, _L = 2, 16, 16
_NW = _NC * _NS
_PER_W = _N // _NW
_CHUNK = 128
_NCHUNK = _PER_W // _CHUNK
_SCALE = math.sqrt(_EMB)


def _body(tok_hbm, table_hbm, out_hbm, idx_v, rows_v, sem):
    wid = lax.axis_index("s") * _NC + lax.axis_index("c")
    base = wid * _PER_W
    pltpu.sync_copy(tok_hbm.at[wid], idx_v)
    for j in range(_NCHUNK):
        pltpu.async_copy(table_hbm.at[idx_v.at[j]], rows_v, sem).wait()

        def _mul_row(i, _):
            for k in range(_EMB // _L):
                sl = (i, pl.ds(k * _L, _L))
                rows_v[sl] = rows_v[sl] * _SCALE
            return 0

        lax.fori_loop(0, _CHUNK, _mul_row, 0)
        pltpu.sync_copy(rows_v, out_hbm.at[pl.ds(base + j * _CHUNK, _CHUNK)])


@jax.jit
def kernel(tokens, table):
    tok = tokens.astype(jnp.int32).reshape(_NW, _NCHUNK, _CHUNK)
    mesh = plsc.VectorSubcoreMesh(core_axis_name="c", subcore_axis_name="s")
    out = pl.kernel(
        _body,
        out_type=jax.ShapeDtypeStruct((_N, _EMB), jnp.float32),
        mesh=mesh,
        scratch_types=[
            pltpu.VMEM((_NCHUNK, _CHUNK), jnp.int32),
            pltpu.VMEM((_CHUNK, _EMB), jnp.float32),
            pltpu.SemaphoreType.DMA,
        ],
    )(tok, table)
    return out.reshape(_SEQ, _BATCH, _EMB)


# speedup vs baseline: 7.7778x; 1.6675x over previous
import math

import jax
import jax.numpy as jnp
from jax import lax
from jax.experimental import pallas as pl
from jax.experimental.pallas import tpu as pltpu
from jax.experimental.pallas import tpu_sc as plsc

_SEQ, _BATCH, _EMB = 200, 1024, 128
_N = _SEQ * _BATCH
_NC, _NS, _L = 2, 16, 16
_NW = _NC * _NS
_PER_W = _N // _NW
_CHUNK = 128
_GPS = 2
_ROWS = _CHUNK * _GPS
_NSTEP = _PER_W // _ROWS
_SCALE = math.sqrt(_EMB)


def _body(tok_hbm, table_hbm, out_hbm, idx_v, rows0, rows1, gs0, gs1, ss0, ss1):
    wid = lax.axis_index("s") * _NC + lax.axis_index("c")
    base = wid * _PER_W
    pltpu.sync_copy(tok_hbm.at[wid], idx_v)

    bufs = (rows0, rows1)
    gsems = (gs0, gs1)
    ssems = (ss0, ss1)

    def start_gathers(step, buf, gsem):
        for g in range(_GPS):
            pltpu.async_copy(
                table_hbm.at[idx_v.at[step * _GPS + g]],
                buf.at[pl.ds(g * _CHUNK, _CHUNK)],
                gsem,
            )

    def scale(buf):
        def _mul_row(i, _):
            for k in range(_EMB // _L):
                sl = (i, pl.ds(k * _L, _L))
                buf[sl] = buf[sl] * _SCALE
            return 0

        lax.fori_loop(0, _ROWS, _mul_row, 0)

    start_gathers(0, bufs[0], gsems[0])
    scat = [None, None]
    for j in range(_NSTEP):
        b = j % 2
        nb = 1 - b
        if j + 1 < _NSTEP:
            if scat[nb] is not None:
                scat[nb].wait()
            start_gathers(j + 1, bufs[nb], gsems[nb])
        pltpu.make_async_copy(
            table_hbm.at[idx_v.at[0]], bufs[b].at[pl.ds(0, _CHUNK)], gsems[b]
        ).wait()
        pltpu.make_async_copy(
            table_hbm.at[idx_v.at[0]], bufs[b].at[pl.ds(_CHUNK, _CHUNK)], gsems[b]
        ).wait()
        scale(bufs[b])
        scat[b] = pltpu.async_copy(
            bufs[b], out_hbm.at[pl.ds(base + j * _ROWS, _ROWS)], ssems[b]
        )
    for h in scat:
        if h is not None:
            h.wait()


@jax.jit
def kernel(tokens, table):
    tok = tokens.astype(jnp.int32).reshape(_NW, _PER_W // _CHUNK, _CHUNK)
    mesh = plsc.VectorSubcoreMesh(core_axis_name="c", subcore_axis_name="s")
    out = pl.kernel(
        _body,
        out_type=jax.ShapeDtypeStruct((_N, _EMB), jnp.float32),
        mesh=mesh,
        scratch_types=[
            pltpu.VMEM((_PER_W // _CHUNK, _CHUNK), jnp.int32),
            pltpu.VMEM((_ROWS, _EMB), jnp.float32),
            pltpu.VMEM((_ROWS, _EMB), jnp.float32),
            pltpu.SemaphoreType.DMA,
            pltpu.SemaphoreType.DMA,
            pltpu.SemaphoreType.DMA,
            pltpu.SemaphoreType.DMA,
        ],
    )(tok, table)
    return out.reshape(_SEQ, _BATCH, _EMB)
